# baseline (speedup 1.0000x reference)
import functools
import math

import jax
import jax.numpy as jnp
from jax import lax
from jax.experimental import pallas as pl
from jax.experimental.pallas import tpu as pltpu
from jax.experimental.pallas import tpu_sc as plsc

D_MODEL = 64
SCALE = math.sqrt(D_MODEL)

NC = 2
NS = 16
NW = NC * NS

B_TOTAL = 4096 * 200
B_PER_W = B_TOTAL // NW
CHUNK = 128
N_CHUNKS = B_PER_W // CHUNK


def _emb_body(x_hbm, table_hbm, out_hbm, idx_v, rows_v, sem):
    wid = lax.axis_index("s") * NC + lax.axis_index("c")
    base = wid * B_PER_W

    pltpu.sync_copy(x_hbm.at[wid], idx_v)

    def chunk_step(j, _):
        pltpu.async_copy(table_hbm.at[idx_v.at[j]], rows_v, sem).wait()

        def scale_row(r, _):
            for i in range(D_MODEL // 16):
                sl = pl.ds(i * 16, 16)
                rows_v[r, sl] = rows_v[r, sl] * SCALE
            return 0

        lax.fori_loop(0, CHUNK, scale_row, 0)

        pltpu.sync_copy(rows_v, out_hbm.at[pl.ds(base + j * CHUNK, CHUNK)])
        return 0

    lax.fori_loop(0, N_CHUNKS, chunk_step, 0)


def _emb(x3, table):
    mesh = plsc.VectorSubcoreMesh(core_axis_name="c", subcore_axis_name="s")
    k = functools.partial(
        pl.kernel,
        mesh=mesh,
        out_type=jax.ShapeDtypeStruct((B_TOTAL, D_MODEL), jnp.float32),
        scratch_types=[
            pltpu.VMEM((N_CHUNKS, CHUNK), jnp.int32),
            pltpu.VMEM((CHUNK, D_MODEL), jnp.float32),
            pltpu.SemaphoreType.DMA,
        ],
        compiler_params=pltpu.CompilerParams(use_tc_tiling_on_sc=False),
    )(_emb_body)
    return k(x3, table)


def kernel(x, table):
    x3 = x.astype(jnp.int32).reshape(NW, N_CHUNKS, CHUNK)
    out = _emb(x3, table)
    return out.reshape(4096, 200, D_MODEL)


# speedup vs baseline: 1.2111x; 1.2111x over previous
import functools
import math

import jax
import jax.numpy as jnp
from jax import lax
from jax.experimental import pallas as pl
from jax.experimental.pallas import tpu as pltpu
from jax.experimental.pallas import tpu_sc as plsc

D = 64
SCALE = math.sqrt(D)

NC = 2
NS = 16
NW = NC * NS

X_ROWS = 4096
X_COLS = 200
ROWS_PER_W = X_ROWS // NW
G1 = 128
G2 = X_COLS - G1
NBUF_G = 4
NBUF_O = 2
N_STEADY = ROWS_PER_W // NBUF_G


def _emb_body(x_hbm, table_hbm, out_hbm, idx_v,
              g0, g1, g2, g3, o0, o1,
              in_s0, in_s1, in_s2, in_s3, out_s0, out_s1):
    gbufs = [g0, g1, g2, g3]
    obufs = [o0, o1]
    in_sems = [in_s0, in_s1, in_s2, in_s3]
    out_sems = [out_s0, out_s1]

    wid = lax.axis_index("s") * NC + lax.axis_index("c")
    base = wid * ROWS_PER_W

    pltpu.sync_copy(x_hbm.at[pl.ds(base, ROWS_PER_W)], idx_v)

    def fire_gather(i, b):
        pltpu.async_copy(
            table_hbm.at[idx_v.at[i, pl.ds(0, G1)]],
            gbufs[b].at[pl.ds(0, G1)], in_sems[b])
        pltpu.async_copy(
            table_hbm.at[idx_v.at[i, pl.ds(G1, G2)]],
            gbufs[b].at[pl.ds(G1, G2)], in_sems[b])

    def wait_gather(b):
        pltpu.make_async_copy(
            table_hbm.at[pl.ds(0, G1)], gbufs[b].at[pl.ds(0, G1)],
            in_sems[b]).wait()
        pltpu.make_async_copy(
            table_hbm.at[pl.ds(0, G2)], gbufs[b].at[pl.ds(G1, G2)],
            in_sems[b]).wait()

    def scale(b, ob):
        gbuf, obuf = gbufs[b], obufs[ob]

        @plsc.parallel_loop(0, X_COLS, 1, unroll=4)
        def _(r):
            for k in range(D // 16):
                sl = pl.ds(k * 16, 16)
                obuf[r, sl] = gbuf[r, sl] * SCALE

    def fire_write(i, ob):
        pltpu.async_copy(obufs[ob], out_hbm.at[base + i], out_sems[ob])

    def wait_write(ob):
        pltpu.make_async_copy(obufs[ob], out_hbm.at[0], out_sems[ob]).wait()

    for b in range(NBUF_G):
        fire_gather(b, b)

    for b in range(NBUF_G):
        wait_gather(b)
        if b >= NBUF_O:
            wait_write(b % NBUF_O)
        scale(b, b % NBUF_O)
        fire_write(b, b % NBUF_O)
        fire_gather(b + NBUF_G, b)

    def outer(g, _):
        for b in range(NBUF_G):
            i = g * NBUF_G + b
            wait_gather(b)
            wait_write(b % NBUF_O)
            scale(b, b % NBUF_O)
            fire_write(i, b % NBUF_O)
            fire_gather(i + NBUF_G, b)
        return 0

    lax.fori_loop(1, N_STEADY - 1, outer, 0)

    for b in range(NBUF_G):
        i = (N_STEADY - 1) * NBUF_G + b
        wait_gather(b)
        wait_write(b % NBUF_O)
        scale(b, b % NBUF_O)
        fire_write(i, b % NBUF_O)

    for ob in range(NBUF_O):
        wait_write(ob)


def _emb(x, table):
    mesh = plsc.VectorSubcoreMesh(core_axis_name="c", subcore_axis_name="s")
    buf = pltpu.VMEM((X_COLS, D), jnp.float32)
    k = functools.partial(
        pl.kernel,
        mesh=mesh,
        out_type=jax.ShapeDtypeStruct((X_ROWS, X_COLS, D), jnp.float32),
        scratch_types=(
            [pltpu.VMEM((ROWS_PER_W, X_COLS), jnp.int32)]
            + [buf] * (NBUF_G + NBUF_O)
            + [pltpu.SemaphoreType.DMA] * (NBUF_G + NBUF_O)
        ),
        compiler_params=pltpu.CompilerParams(use_tc_tiling_on_sc=False),
    )(_emb_body)
    return k(x, table)


def kernel(x, table):
    return _emb(x.astype(jnp.int32), table)
